# baseline (speedup 1.0000x reference)
import dataclasses
import functools

import jax
import jax.numpy as jnp
from jax import lax
from jax.experimental import pallas as pl
from jax.experimental.pallas import tpu as pltpu
from jax.experimental.pallas import tpu_sc as plsc

V = 1000000
D = 64
B = 16384
C = 20
NNEG = 20

NC = 2
NS = 16
NW = NC * NS
BPW = B // NW
BK = 16
NCHUNK = BPW // BK
ROWS = BK * C
SUB = 64
NSUB = ROWS // SUB


def _row_f32(ref, r):
    h0 = plsc.unpack(ref[r, pl.ds(0, 32)], format=plsc.PackFormat.INTERLEAVED)
    h1 = plsc.unpack(ref[r, pl.ds(32, 32)], format=plsc.PackFormat.INTERLEAVED)
    return [h0[0], h0[1], h1[0], h1[1]]


def _sc_ctx_body(emb_hbm, ctx_idx_hbm, sums_hbm,
                 ctx_idx_v, ctx_rows0, ctx_rows1, sums_acc, sem0, sem1):
    wid = lax.axis_index("s") * NC + lax.axis_index("c")
    pltpu.sync_copy(ctx_idx_hbm.at[pl.ds(wid * (BPW * C // SUB),
                                         BPW * C // SUB)], ctx_idx_v)
    bufs = ((ctx_rows0, sem0), (ctx_rows1, sem1))

    def fire(c, par):
        ctx_rows, sem = bufs[par]
        for j in range(NSUB):
            pltpu.async_copy(emb_hbm.at[ctx_idx_v.at[c * NSUB + j]],
                             ctx_rows.at[pl.ds(j * SUB, SUB)], sem)

    def drain(c, par):
        ctx_rows, sem = bufs[par]
        for j in range(NSUB):
            pltpu.make_async_copy(emb_hbm.at[ctx_idx_v.at[c * NSUB + j]],
                                  ctx_rows.at[pl.ds(j * SUB, SUB)], sem).wait()

    def compute(c, par):
        ctx_rows, _ = bufs[par]

        @pl.loop(0, BK)
        def _(b):
            m = _row_f32(ctx_rows, b * C)
            for i in range(1, C):
                row = _row_f32(ctx_rows, b * C + i)
                for k in range(4):
                    m[k] = m[k] + row[k]
            for k in range(4):
                sums_acc[c * BK + b, pl.ds(k * 16, 16)] = m[k]

    fire(0, 0)

    @pl.loop(0, NCHUNK, step=2)
    def _(c):
        fire(c + 1, 1)
        drain(c, 0)
        compute(c, 0)

        @pl.when(c + 2 < NCHUNK)
        def _():
            fire(c + 2, 0)

        drain(c + 1, 1)
        compute(c + 1, 1)

    pltpu.sync_copy(sums_acc, sums_hbm.at[pl.ds(wid * BPW, BPW)])


BK2 = 32
NCHUNK2 = BPW // BK2
ROWS2 = BK2 * NNEG
SUB2 = 128
NSUB2 = ROWS2 // SUB2


def _sc_dots_body(ctxw_hbm, sums_hbm, tgt_idx_hbm, neg_idx_hbm,
                  pos_hbm, negs_hbm,
                  neg_idx_v, tgt_idx_v,
                  neg_rows0, tgt_rows0, sums0, neg_rows1, tgt_rows1, sums1,
                  pos_acc, neg_acc, sem0, sem1):
    wid = lax.axis_index("s") * NC + lax.axis_index("c")
    pltpu.sync_copy(neg_idx_hbm.at[pl.ds(wid * (BPW * NNEG // SUB2),
                                         BPW * NNEG // SUB2)], neg_idx_v)
    pltpu.sync_copy(tgt_idx_hbm.at[wid], tgt_idx_v)

    lanes = lax.iota(jnp.int32, 16)
    bufs = ((neg_rows0, tgt_rows0, sums0, sem0),
            (neg_rows1, tgt_rows1, sums1, sem1))

    def fire(c, par):
        neg_rows, tgt_rows, sums_b, sem = bufs[par]
        for j in range(NSUB2):
            pltpu.async_copy(ctxw_hbm.at[neg_idx_v.at[c * NSUB2 + j]],
                             neg_rows.at[pl.ds(j * SUB2, SUB2)], sem)
        pltpu.async_copy(ctxw_hbm.at[tgt_idx_v.at[c]], tgt_rows, sem)
        pltpu.async_copy(sums_hbm.at[pl.ds(wid * BPW + c * BK2, BK2)],
                         sums_b, sem)

    def drain(c, par):
        neg_rows, tgt_rows, sums_b, sem = bufs[par]
        for j in range(NSUB2):
            pltpu.make_async_copy(ctxw_hbm.at[neg_idx_v.at[c * NSUB2 + j]],
                                  neg_rows.at[pl.ds(j * SUB2, SUB2)],
                                  sem).wait()
        pltpu.make_async_copy(ctxw_hbm.at[tgt_idx_v.at[c]], tgt_rows,
                              sem).wait()
        pltpu.make_async_copy(sums_hbm.at[pl.ds(wid * BPW + c * BK2, BK2)],
                              sums_b, sem).wait()

    def compute(c, par):
        neg_rows, tgt_rows, sums_b, _ = bufs[par]

        @pl.loop(0, BK2)
        def _(b):
            m = [sums_b[b, pl.ds(k * 16, 16)] for k in range(4)]
            trow = _row_f32(tgt_rows, b)
            acc = m[0] * trow[0]
            for k in range(1, 4):
                acc = acc + m[k] * trow[k]
            s = jnp.sum(acc)
            p = c * BK2 + b
            pos_acc[p // 16, :] = jnp.where(lanes == p % 16, s,
                                            pos_acc[p // 16, :])
            for n in range(NNEG):
                r = b * NNEG + n
                nrow_v = _row_f32(neg_rows, r)
                acc = m[0] * nrow_v[0]
                for k in range(1, 4):
                    acc = acc + m[k] * nrow_v[k]
                s = jnp.sum(acc)
                g = c * ROWS2 + r
                nrow = g // 16
                nlane = g % 16
                neg_acc[nrow, :] = jnp.where(lanes == nlane, s,
                                             neg_acc[nrow, :])

    fire(0, 0)

    @pl.loop(0, NCHUNK2, step=2)
    def _(c):
        fire(c + 1, 1)
        drain(c, 0)
        compute(c, 0)

        @pl.when(c + 2 < NCHUNK2)
        def _():
            fire(c + 2, 0)

        drain(c + 1, 1)
        compute(c + 1, 1)

    pltpu.sync_copy(pos_acc, pos_hbm.at[pl.ds(wid * (BPW // 16), BPW // 16)])
    pltpu.sync_copy(neg_acc,
                    negs_hbm.at[pl.ds(wid * (BPW * NNEG // 16),
                                      BPW * NNEG // 16)])


_sc_cp = pltpu.CompilerParams()
if "needs_layout_passes" in pltpu.CompilerParams.__dataclass_fields__:
    _sc_cp = dataclasses.replace(_sc_cp, needs_layout_passes=False)
if "use_tc_tiling_on_sc" in pltpu.CompilerParams.__dataclass_fields__:
    _sc_cp = dataclasses.replace(_sc_cp, use_tc_tiling_on_sc=False)

_sc_mesh = plsc.VectorSubcoreMesh(core_axis_name="c", subcore_axis_name="s")

_sc_ctx = functools.partial(
    pl.kernel,
    compiler_params=_sc_cp,
    out_type=jax.ShapeDtypeStruct((B, D), jnp.float32),
    mesh=_sc_mesh,
    scratch_types=[
        pltpu.VMEM((BPW * C // SUB, SUB), jnp.int32),
        pltpu.VMEM((ROWS, D), jnp.bfloat16),
        pltpu.VMEM((ROWS, D), jnp.bfloat16),
        pltpu.VMEM((BPW, D), jnp.float32),
        pltpu.SemaphoreType.DMA,
        pltpu.SemaphoreType.DMA,
    ],
)(_sc_ctx_body)

_sc_dots = functools.partial(
    pl.kernel,
    compiler_params=_sc_cp,
    out_type=(jax.ShapeDtypeStruct((B // 16, 16), jnp.float32),
              jax.ShapeDtypeStruct((B * NNEG // 16, 16), jnp.float32)),
    mesh=_sc_mesh,
    scratch_types=[
        pltpu.VMEM((BPW * NNEG // SUB2, SUB2), jnp.int32),
        pltpu.VMEM((NCHUNK2, BK2), jnp.int32),
        pltpu.VMEM((ROWS2, D), jnp.bfloat16),
        pltpu.VMEM((BK2, D), jnp.bfloat16),
        pltpu.VMEM((BK2, D), jnp.float32),
        pltpu.VMEM((ROWS2, D), jnp.bfloat16),
        pltpu.VMEM((BK2, D), jnp.bfloat16),
        pltpu.VMEM((BK2, D), jnp.float32),
        pltpu.VMEM((BPW // 16, 16), jnp.float32),
        pltpu.VMEM((BPW * NNEG // 16, 16), jnp.float32),
        pltpu.SemaphoreType.DMA,
        pltpu.SemaphoreType.DMA,
    ],
)(_sc_dots_body)


TRBLK = 32768
TRGRID = pl.cdiv(V, TRBLK)
TV = TRGRID * TRBLK


TRSH = 15


def _remap(t):
    blk = t >> TRSH
    w = t & (TRBLK - 1)
    return (blk << TRSH) | ((w & (TRBLK // 2 - 1)) << 1) | (w >> (TRSH - 1))


def _tr_body(in_ref, o_ref):
    xT = in_ref[...].astype(jnp.bfloat16).T
    o_ref[...] = jnp.concatenate([xT[0:TRBLK // 2], xT[TRBLK // 2:]], axis=1)


_transpose = pl.pallas_call(
    _tr_body,
    grid=(TRGRID,),
    in_specs=[pl.BlockSpec((D, TRBLK), lambda i: (0, i))],
    out_specs=pl.BlockSpec((TRBLK // 2, 128), lambda i: (i, 0)),
    out_shape=jax.ShapeDtypeStruct((TV // 2, 128), jnp.bfloat16),
    compiler_params=pltpu.CompilerParams(
        dimension_semantics=("parallel",)),
)


def _tr_idx_body(in_ref, o_ref):
    o_ref[...] = _remap(in_ref[...].T)


_transpose_idx = pl.pallas_call(
    _tr_idx_body,
    grid=(8,),
    in_specs=[pl.BlockSpec((C, B // 8), lambda i: (0, i))],
    out_specs=pl.BlockSpec((B // 8, C), lambda i: (i, 0)),
    out_shape=jax.ShapeDtypeStruct((B, C), jnp.int32),
    compiler_params=pltpu.CompilerParams(
        dimension_semantics=("parallel",)),
)


def _loss_body(pos_ref, neg_ref, o_ref):
    inv_c = jnp.float32(1.0 / C)

    def ls(x):
        return jnp.minimum(x, 0.0) - jnp.log1p(jnp.exp(-jnp.abs(x)))

    pos = pos_ref[...] * inv_c
    neg = neg_ref[...] * inv_c
    total = jnp.sum(ls(pos)) + jnp.sum(ls(-neg))
    o_ref[0, 0] = -(total / jnp.float32(B))


_loss = pl.pallas_call(
    _loss_body,
    out_shape=jax.ShapeDtypeStruct((1, 1), jnp.float32),
    out_specs=pl.BlockSpec(memory_space=pltpu.SMEM),
)


def kernel(context_words, target_word, negative_samples, emb_weight, ctx_weight):
    ctx_idx = _transpose_idx(context_words.astype(jnp.int32).T)
    ctx_idx = ctx_idx.reshape(B * C // SUB, SUB)
    neg_idx = _transpose_idx(negative_samples.astype(jnp.int32).T)
    neg_idx = neg_idx.reshape(B * NNEG // SUB2, SUB2)
    tgt_idx = _remap(target_word.astype(jnp.int32)).reshape(NW, NCHUNK2, BK2)
    emb_lin = _transpose(emb_weight.T).reshape(TV, D)
    ctxw_lin = _transpose(ctx_weight.T).reshape(TV, D)
    sums = _sc_ctx(emb_lin, ctx_idx)
    pos_raw, neg_raw = _sc_dots(ctxw_lin, sums, tgt_idx, neg_idx)
    loss = _loss(pos_raw.reshape(128, 128), neg_raw.reshape(2560, 128))
    return loss[0, 0]


# speedup vs baseline: 1.3489x; 1.3489x over previous
import dataclasses
import functools

import jax
import jax.numpy as jnp
from jax import lax
from jax.experimental import pallas as pl
from jax.experimental.pallas import tpu as pltpu
from jax.experimental.pallas import tpu_sc as plsc

V = 1000000
D = 64
B = 16384
C = 20
NNEG = 20

NC = 2
NS = 16
NW = NC * NS
BPW = B // NW
BK = 16
NCHUNK = BPW // BK
ROWS = BK * C
SUB = 64
NSUB = ROWS // SUB


def _row_f32(ref, r):
    h0 = plsc.bitcast(ref[r, pl.ds(0, 16)], jnp.bfloat16)
    h1 = plsc.bitcast(ref[r, pl.ds(16, 16)], jnp.bfloat16)
    a0, b0 = plsc.unpack(h0, format=plsc.PackFormat.INTERLEAVED)
    a1, b1 = plsc.unpack(h1, format=plsc.PackFormat.INTERLEAVED)
    return [a0, b0, a1, b1]


def _sc_ctx_body(emb_hbm, ctx_idx_hbm, sums_hbm,
                 ctx_idx_v, ctx_rows0, ctx_rows1, sums_acc, sem0, sem1):
    wid = lax.axis_index("s") * NC + lax.axis_index("c")
    pltpu.sync_copy(ctx_idx_hbm.at[pl.ds(wid * (BPW * C // SUB),
                                         BPW * C // SUB)], ctx_idx_v)
    bufs = ((ctx_rows0, sem0), (ctx_rows1, sem1))

    def fire(c, par):
        ctx_rows, sem = bufs[par]
        for j in range(NSUB):
            pltpu.async_copy(emb_hbm.at[ctx_idx_v.at[c * NSUB + j]],
                             ctx_rows.at[pl.ds(j * SUB, SUB)], sem)

    def drain(c, par):
        ctx_rows, sem = bufs[par]
        for j in range(NSUB):
            pltpu.make_async_copy(emb_hbm.at[ctx_idx_v.at[c * NSUB + j]],
                                  ctx_rows.at[pl.ds(j * SUB, SUB)], sem).wait()

    def compute(c, par):
        ctx_rows, _ = bufs[par]

        @pl.loop(0, BK)
        def _(b):
            m = _row_f32(ctx_rows, b * C)
            for i in range(1, C):
                row = _row_f32(ctx_rows, b * C + i)
                for k in range(4):
                    m[k] = m[k] + row[k]
            for k in range(4):
                sums_acc[c * BK + b, pl.ds(k * 16, 16)] = m[k]

    fire(0, 0)

    @pl.loop(0, NCHUNK, step=2)
    def _(c):
        fire(c + 1, 1)
        drain(c, 0)
        compute(c, 0)

        @pl.when(c + 2 < NCHUNK)
        def _():
            fire(c + 2, 0)

        drain(c + 1, 1)
        compute(c + 1, 1)

    pltpu.sync_copy(sums_acc, sums_hbm.at[pl.ds(wid * BPW, BPW)])


BK2 = 32
NCHUNK2 = BPW // BK2
ROWS2 = BK2 * NNEG
SUB2 = 128
NSUB2 = ROWS2 // SUB2


def _sc_dots_body(ctxw_hbm, sums_hbm, tgt_idx_hbm, neg_idx_hbm,
                  pos_hbm, negs_hbm,
                  neg_idx_v, tgt_idx_v,
                  neg_rows0, tgt_rows0, sums0, neg_rows1, tgt_rows1, sums1,
                  pos_acc, neg_acc, sem0, sem1):
    wid = lax.axis_index("s") * NC + lax.axis_index("c")
    pltpu.sync_copy(neg_idx_hbm.at[pl.ds(wid * (BPW * NNEG // SUB2),
                                         BPW * NNEG // SUB2)], neg_idx_v)
    pltpu.sync_copy(tgt_idx_hbm.at[wid], tgt_idx_v)

    lanes = lax.iota(jnp.int32, 16)
    bufs = ((neg_rows0, tgt_rows0, sums0, sem0),
            (neg_rows1, tgt_rows1, sums1, sem1))

    def fire(c, par):
        neg_rows, tgt_rows, sums_b, sem = bufs[par]
        for j in range(NSUB2):
            pltpu.async_copy(ctxw_hbm.at[neg_idx_v.at[c * NSUB2 + j]],
                             neg_rows.at[pl.ds(j * SUB2, SUB2)], sem)
        pltpu.async_copy(ctxw_hbm.at[tgt_idx_v.at[c]], tgt_rows, sem)
        pltpu.async_copy(sums_hbm.at[pl.ds(wid * BPW + c * BK2, BK2)],
                         sums_b, sem)

    def drain(c, par):
        neg_rows, tgt_rows, sums_b, sem = bufs[par]
        for j in range(NSUB2):
            pltpu.make_async_copy(ctxw_hbm.at[neg_idx_v.at[c * NSUB2 + j]],
                                  neg_rows.at[pl.ds(j * SUB2, SUB2)],
                                  sem).wait()
        pltpu.make_async_copy(ctxw_hbm.at[tgt_idx_v.at[c]], tgt_rows,
                              sem).wait()
        pltpu.make_async_copy(sums_hbm.at[pl.ds(wid * BPW + c * BK2, BK2)],
                              sums_b, sem).wait()

    def compute(c, par):
        neg_rows, tgt_rows, sums_b, _ = bufs[par]

        @pl.loop(0, BK2)
        def _(b):
            m = [sums_b[b, pl.ds(k * 16, 16)] for k in range(4)]
            trow = _row_f32(tgt_rows, b)
            acc = m[0] * trow[0]
            for k in range(1, 4):
                acc = acc + m[k] * trow[k]
            s = jnp.sum(acc)
            p = c * BK2 + b
            pos_acc[p // 16, :] = jnp.where(lanes == p % 16, s,
                                            pos_acc[p // 16, :])
            for n in range(NNEG):
                r = b * NNEG + n
                nrow_v = _row_f32(neg_rows, r)
                acc = m[0] * nrow_v[0]
                for k in range(1, 4):
                    acc = acc + m[k] * nrow_v[k]
                s = jnp.sum(acc)
                g = c * ROWS2 + r
                nrow = g // 16
                nlane = g % 16
                neg_acc[nrow, :] = jnp.where(lanes == nlane, s,
                                             neg_acc[nrow, :])

    fire(0, 0)

    @pl.loop(0, NCHUNK2, step=2)
    def _(c):
        fire(c + 1, 1)
        drain(c, 0)
        compute(c, 0)

        @pl.when(c + 2 < NCHUNK2)
        def _():
            fire(c + 2, 0)

        drain(c + 1, 1)
        compute(c + 1, 1)

    pltpu.sync_copy(pos_acc, pos_hbm.at[pl.ds(wid * (BPW // 16), BPW // 16)])
    pltpu.sync_copy(neg_acc,
                    negs_hbm.at[pl.ds(wid * (BPW * NNEG // 16),
                                      BPW * NNEG // 16)])


_sc_cp = pltpu.CompilerParams()
if "needs_layout_passes" in pltpu.CompilerParams.__dataclass_fields__:
    _sc_cp = dataclasses.replace(_sc_cp, needs_layout_passes=False)
if "use_tc_tiling_on_sc" in pltpu.CompilerParams.__dataclass_fields__:
    _sc_cp = dataclasses.replace(_sc_cp, use_tc_tiling_on_sc=False)

_sc_mesh = plsc.VectorSubcoreMesh(core_axis_name="c", subcore_axis_name="s")

_sc_ctx = functools.partial(
    pl.kernel,
    compiler_params=_sc_cp,
    out_type=jax.ShapeDtypeStruct((B, D), jnp.float32),
    mesh=_sc_mesh,
    scratch_types=[
        pltpu.VMEM((BPW * C // SUB, SUB), jnp.int32),
        pltpu.VMEM((ROWS, 32), jnp.float32),
        pltpu.VMEM((ROWS, 32), jnp.float32),
        pltpu.VMEM((BPW, D), jnp.float32),
        pltpu.SemaphoreType.DMA,
        pltpu.SemaphoreType.DMA,
    ],
)(_sc_ctx_body)

_sc_dots = functools.partial(
    pl.kernel,
    compiler_params=_sc_cp,
    out_type=(jax.ShapeDtypeStruct((B // 16, 16), jnp.float32),
              jax.ShapeDtypeStruct((B * NNEG // 16, 16), jnp.float32)),
    mesh=_sc_mesh,
    scratch_types=[
        pltpu.VMEM((BPW * NNEG // SUB2, SUB2), jnp.int32),
        pltpu.VMEM((NCHUNK2, BK2), jnp.int32),
        pltpu.VMEM((ROWS2, 32), jnp.float32),
        pltpu.VMEM((BK2, 32), jnp.float32),
        pltpu.VMEM((BK2, D), jnp.float32),
        pltpu.VMEM((ROWS2, 32), jnp.float32),
        pltpu.VMEM((BK2, 32), jnp.float32),
        pltpu.VMEM((BK2, D), jnp.float32),
        pltpu.VMEM((BPW // 16, 16), jnp.float32),
        pltpu.VMEM((BPW * NNEG // 16, 16), jnp.float32),
        pltpu.SemaphoreType.DMA,
        pltpu.SemaphoreType.DMA,
    ],
)(_sc_dots_body)


TRBLK = 32768
TRGRID = pl.cdiv(V, TRBLK)
TV = TRGRID * TRBLK


TRSH = 15


def _remap(t):
    blk = t >> TRSH
    w = t & (TRBLK - 1)
    ql = w & (TRBLK // 4 - 1)
    r = w >> (TRSH - 2)
    return (blk << TRSH) | (ql << 2) | r


def _rtne16(u):
    rnd = jnp.int32(0x7FFF) + jnp.bitwise_and(
        jax.lax.shift_right_logical(u, 16), jnp.int32(1))
    return jnp.bitwise_and(jax.lax.shift_right_logical(u + rnd, 16),
                           jnp.int32(0xFFFF))


def _tr_body(in_ref, o_ref):
    xT = in_ref[...].T
    ua = jax.lax.bitcast_convert_type(xT[:, 0:32], jnp.int32)
    ub = jax.lax.bitcast_convert_type(xT[:, 32:64], jnp.int32)
    w = jnp.bitwise_or(jax.lax.shift_left(_rtne16(ub), 16), _rtne16(ua))
    wf = jax.lax.bitcast_convert_type(w, jnp.float32)
    q = TRBLK // 4
    o_ref[...] = jnp.concatenate(
        [wf[0:q], wf[q:2 * q], wf[2 * q:3 * q], wf[3 * q:]], axis=1)


_transpose = pl.pallas_call(
    _tr_body,
    grid=(TRGRID,),
    in_specs=[pl.BlockSpec((D, TRBLK), lambda i: (0, i))],
    out_specs=pl.BlockSpec((TRBLK // 4, 128), lambda i: (i, 0)),
    out_shape=jax.ShapeDtypeStruct((TV // 4, 128), jnp.float32),
    compiler_params=pltpu.CompilerParams(
        dimension_semantics=("parallel",)),
)


def _tr_idx_body(in_ref, o_ref):
    o_ref[...] = _remap(in_ref[...].T)


_transpose_idx = pl.pallas_call(
    _tr_idx_body,
    grid=(8,),
    in_specs=[pl.BlockSpec((C, B // 8), lambda i: (0, i))],
    out_specs=pl.BlockSpec((B // 8, C), lambda i: (i, 0)),
    out_shape=jax.ShapeDtypeStruct((B, C), jnp.int32),
    compiler_params=pltpu.CompilerParams(
        dimension_semantics=("parallel",)),
)


def _loss_body(pos_ref, neg_ref, o_ref):
    inv_c = jnp.float32(1.0 / C)

    def ls(x):
        return jnp.minimum(x, 0.0) - jnp.log1p(jnp.exp(-jnp.abs(x)))

    pos = pos_ref[...] * inv_c
    neg = neg_ref[...] * inv_c
    total = jnp.sum(ls(pos)) + jnp.sum(ls(-neg))
    o_ref[0, 0] = -(total / jnp.float32(B))


_loss = pl.pallas_call(
    _loss_body,
    out_shape=jax.ShapeDtypeStruct((1, 1), jnp.float32),
    out_specs=pl.BlockSpec(memory_space=pltpu.SMEM),
)


def kernel(context_words, target_word, negative_samples, emb_weight, ctx_weight):
    ctx_idx = _transpose_idx(context_words.astype(jnp.int32).T)
    ctx_idx = ctx_idx.reshape(B * C // SUB, SUB)
    neg_idx = _transpose_idx(negative_samples.astype(jnp.int32).T)
    neg_idx = neg_idx.reshape(B * NNEG // SUB2, SUB2)
    tgt_idx = _remap(target_word.astype(jnp.int32)).reshape(NW, NCHUNK2, BK2)
    emb_lin = _transpose(emb_weight.T).reshape(TV, 32)
    ctxw_lin = _transpose(ctx_weight.T).reshape(TV, 32)
    sums = _sc_ctx(emb_lin, ctx_idx)
    pos_raw, neg_raw = _sc_dots(ctxw_lin, sums, tgt_idx, neg_idx)
    loss = _loss(pos_raw.reshape(128, 128), neg_raw.reshape(2560, 128))
    return loss[0, 0]


# speedup vs baseline: 2.0012x; 1.4836x over previous
import dataclasses
import functools

import jax
import jax.numpy as jnp
from jax import lax
from jax.experimental import pallas as pl
from jax.experimental.pallas import tpu as pltpu
from jax.experimental.pallas import tpu_sc as plsc

V = 1000000
D = 64
B = 16384
C = 20
NNEG = 20

NC = 2
NS = 16
NW = NC * NS
BPW = B // NW
BK = 16
NCHUNK = BPW // BK
ROWS = BK * C
SUB = 64
NSUB = ROWS // SUB


def _sc_ctx_body(emb_hbm, ctx_idx_hbm, sums_hbm,
                 ctx_idx_v, ctx_rows0, ctx_rows1, sums_acc, sem0, sem1):
    wid = lax.axis_index("s") * NC + lax.axis_index("c")
    pltpu.sync_copy(ctx_idx_hbm.at[pl.ds(wid * (BPW * C // SUB),
                                         BPW * C // SUB)], ctx_idx_v)
    bufs = ((ctx_rows0, sem0), (ctx_rows1, sem1))

    def fire(c, par):
        ctx_rows, sem = bufs[par]
        for j in range(NSUB):
            pltpu.async_copy(emb_hbm.at[ctx_idx_v.at[c * NSUB + j]],
                             ctx_rows.at[pl.ds(j * SUB, SUB)], sem)

    def drain(c, par):
        ctx_rows, sem = bufs[par]
        for j in range(NSUB):
            pltpu.make_async_copy(emb_hbm.at[ctx_idx_v.at[c * NSUB + j]],
                                  ctx_rows.at[pl.ds(j * SUB, SUB)], sem).wait()

    def compute(c, par):
        ctx_rows, _ = bufs[par]

        @pl.loop(0, BK)
        def _(b):
            m = [ctx_rows[b * C, pl.ds(k * 16, 16)] for k in range(4)]
            for i in range(1, C):
                for k in range(4):
                    m[k] = m[k] + ctx_rows[b * C + i, pl.ds(k * 16, 16)]
            for k in range(4):
                sums_acc[c * BK + b, pl.ds(k * 16, 16)] = m[k]

    fire(0, 0)

    @pl.loop(0, NCHUNK, step=2)
    def _(c):
        fire(c + 1, 1)
        drain(c, 0)
        compute(c, 0)

        @pl.when(c + 2 < NCHUNK)
        def _():
            fire(c + 2, 0)

        drain(c + 1, 1)
        compute(c + 1, 1)

    pltpu.sync_copy(sums_acc, sums_hbm.at[pl.ds(wid * BPW, BPW)])


BK2 = 32
NCHUNK2 = BPW // BK2
ROWS2 = BK2 * NNEG
SUB2 = 128
NSUB2 = ROWS2 // SUB2


def _sc_dots_body(ctxw_hbm, sums_hbm, tgt_idx_hbm, neg_idx_hbm,
                  pos_hbm, negs_hbm,
                  neg_idx_v, tgt_idx_v,
                  neg_rows0, tgt_rows0, sums0, neg_rows1, tgt_rows1, sums1,
                  pos_acc, neg_acc, sem0, sem1):
    wid = lax.axis_index("s") * NC + lax.axis_index("c")
    pltpu.sync_copy(neg_idx_hbm.at[pl.ds(wid * (BPW * NNEG // SUB2),
                                         BPW * NNEG // SUB2)], neg_idx_v)
    pltpu.sync_copy(tgt_idx_hbm.at[wid], tgt_idx_v)

    lanes = lax.iota(jnp.int32, 16)
    bufs = ((neg_rows0, tgt_rows0, sums0, sem0),
            (neg_rows1, tgt_rows1, sums1, sem1))

    def fire(c, par):
        neg_rows, tgt_rows, sums_b, sem = bufs[par]
        for j in range(NSUB2):
            pltpu.async_copy(ctxw_hbm.at[neg_idx_v.at[c * NSUB2 + j]],
                             neg_rows.at[pl.ds(j * SUB2, SUB2)], sem)
        pltpu.async_copy(ctxw_hbm.at[tgt_idx_v.at[c]], tgt_rows, sem)
        pltpu.async_copy(sums_hbm.at[pl.ds(wid * BPW + c * BK2, BK2)],
                         sums_b, sem)

    def drain(c, par):
        neg_rows, tgt_rows, sums_b, sem = bufs[par]
        for j in range(NSUB2):
            pltpu.make_async_copy(ctxw_hbm.at[neg_idx_v.at[c * NSUB2 + j]],
                                  neg_rows.at[pl.ds(j * SUB2, SUB2)],
                                  sem).wait()
        pltpu.make_async_copy(ctxw_hbm.at[tgt_idx_v.at[c]], tgt_rows,
                              sem).wait()
        pltpu.make_async_copy(sums_hbm.at[pl.ds(wid * BPW + c * BK2, BK2)],
                              sums_b, sem).wait()

    def compute(c, par):
        neg_rows, tgt_rows, sums_b, _ = bufs[par]

        @pl.loop(0, BK2)
        def _(b):
            m = [sums_b[b, pl.ds(k * 16, 16)] for k in range(4)]
            acc = m[0] * tgt_rows[b, pl.ds(0, 16)]
            for k in range(1, 4):
                acc = acc + m[k] * tgt_rows[b, pl.ds(k * 16, 16)]
            s = jnp.sum(acc)
            p = c * BK2 + b
            pos_acc[p // 16, :] = jnp.where(lanes == p % 16, s,
                                            pos_acc[p // 16, :])
            for n in range(NNEG):
                r = b * NNEG + n
                acc = m[0] * neg_rows[r, pl.ds(0, 16)]
                for k in range(1, 4):
                    acc = acc + m[k] * neg_rows[r, pl.ds(k * 16, 16)]
                s = jnp.sum(acc)
                g = c * ROWS2 + r
                nrow = g // 16
                nlane = g % 16
                neg_acc[nrow, :] = jnp.where(lanes == nlane, s,
                                             neg_acc[nrow, :])

    fire(0, 0)

    @pl.loop(0, NCHUNK2, step=2)
    def _(c):
        fire(c + 1, 1)
        drain(c, 0)
        compute(c, 0)

        @pl.when(c + 2 < NCHUNK2)
        def _():
            fire(c + 2, 0)

        drain(c + 1, 1)
        compute(c + 1, 1)

    pltpu.sync_copy(pos_acc, pos_hbm.at[pl.ds(wid * (BPW // 16), BPW // 16)])
    pltpu.sync_copy(neg_acc,
                    negs_hbm.at[pl.ds(wid * (BPW * NNEG // 16),
                                      BPW * NNEG // 16)])


_sc_cp = pltpu.CompilerParams()
if "needs_layout_passes" in pltpu.CompilerParams.__dataclass_fields__:
    _sc_cp = dataclasses.replace(_sc_cp, needs_layout_passes=False)
if "use_tc_tiling_on_sc" in pltpu.CompilerParams.__dataclass_fields__:
    _sc_cp = dataclasses.replace(_sc_cp, use_tc_tiling_on_sc=False)

_sc_mesh = plsc.VectorSubcoreMesh(core_axis_name="c", subcore_axis_name="s")

_sc_ctx = functools.partial(
    pl.kernel,
    compiler_params=_sc_cp,
    out_type=jax.ShapeDtypeStruct((B, D), jnp.float32),
    mesh=_sc_mesh,
    scratch_types=[
        pltpu.VMEM((BPW * C // SUB, SUB), jnp.int32),
        pltpu.VMEM((ROWS, D), jnp.float32),
        pltpu.VMEM((ROWS, D), jnp.float32),
        pltpu.VMEM((BPW, D), jnp.float32),
        pltpu.SemaphoreType.DMA,
        pltpu.SemaphoreType.DMA,
    ],
)(_sc_ctx_body)

_sc_dots = functools.partial(
    pl.kernel,
    compiler_params=_sc_cp,
    out_type=(jax.ShapeDtypeStruct((B // 16, 16), jnp.float32),
              jax.ShapeDtypeStruct((B * NNEG // 16, 16), jnp.float32)),
    mesh=_sc_mesh,
    scratch_types=[
        pltpu.VMEM((BPW * NNEG // SUB2, SUB2), jnp.int32),
        pltpu.VMEM((NCHUNK2, BK2), jnp.int32),
        pltpu.VMEM((ROWS2, D), jnp.float32),
        pltpu.VMEM((BK2, D), jnp.float32),
        pltpu.VMEM((BK2, D), jnp.float32),
        pltpu.VMEM((ROWS2, D), jnp.float32),
        pltpu.VMEM((BK2, D), jnp.float32),
        pltpu.VMEM((BK2, D), jnp.float32),
        pltpu.VMEM((BPW // 16, 16), jnp.float32),
        pltpu.VMEM((BPW * NNEG // 16, 16), jnp.float32),
        pltpu.SemaphoreType.DMA,
        pltpu.SemaphoreType.DMA,
    ],
)(_sc_dots_body)


TRBLK = 32768
TRGRID = pl.cdiv(V, TRBLK)
TV = TRGRID * TRBLK


TRSH = 15


def _remap(t):
    blk = t >> TRSH
    w = t & (TRBLK - 1)
    return (blk << TRSH) | ((w & (TRBLK // 2 - 1)) << 1) | (w >> (TRSH - 1))


def _tr_body(in_ref, o_ref):
    xT = in_ref[...].T
    o_ref[...] = jnp.concatenate([xT[0:TRBLK // 2], xT[TRBLK // 2:]], axis=1)


_transpose = pl.pallas_call(
    _tr_body,
    grid=(TRGRID,),
    in_specs=[pl.BlockSpec((D, TRBLK), lambda i: (0, i))],
    out_specs=pl.BlockSpec((TRBLK // 2, 128), lambda i: (i, 0)),
    out_shape=jax.ShapeDtypeStruct((TV // 2, 128), jnp.float32),
    compiler_params=pltpu.CompilerParams(
        dimension_semantics=("parallel",)),
)


def _tr_idx_body(in_ref, o_ref):
    o_ref[...] = _remap(in_ref[...].T)


_transpose_idx = pl.pallas_call(
    _tr_idx_body,
    grid=(8,),
    in_specs=[pl.BlockSpec((C, B // 8), lambda i: (0, i))],
    out_specs=pl.BlockSpec((B // 8, C), lambda i: (i, 0)),
    out_shape=jax.ShapeDtypeStruct((B, C), jnp.int32),
    compiler_params=pltpu.CompilerParams(
        dimension_semantics=("parallel",)),
)


def _loss_body(pos_ref, neg_ref, o_ref):
    inv_c = jnp.float32(1.0 / C)

    def ls(x):
        return jnp.minimum(x, 0.0) - jnp.log1p(jnp.exp(-jnp.abs(x)))

    pos = pos_ref[...] * inv_c
    neg = neg_ref[...] * inv_c
    total = jnp.sum(ls(pos)) + jnp.sum(ls(-neg))
    o_ref[0, 0] = -(total / jnp.float32(B))


_loss = pl.pallas_call(
    _loss_body,
    out_shape=jax.ShapeDtypeStruct((1, 1), jnp.float32),
    out_specs=pl.BlockSpec(memory_space=pltpu.SMEM),
)


def kernel(context_words, target_word, negative_samples, emb_weight, ctx_weight):
    ctx_idx = _transpose_idx(context_words.astype(jnp.int32).T)
    ctx_idx = ctx_idx.reshape(B * C // SUB, SUB)
    neg_idx = _transpose_idx(negative_samples.astype(jnp.int32).T)
    neg_idx = neg_idx.reshape(B * NNEG // SUB2, SUB2)
    tgt_idx = _remap(target_word.astype(jnp.int32)).reshape(NW, NCHUNK2, BK2)
    emb_lin = _transpose(emb_weight.T).reshape(TV, D)
    ctxw_lin = _transpose(ctx_weight.T).reshape(TV, D)
    sums = _sc_ctx(emb_lin, ctx_idx)
    pos_raw, neg_raw = _sc_dots(ctxw_lin, sums, tgt_idx, neg_idx)
    loss = _loss(pos_raw.reshape(128, 128), neg_raw.reshape(2560, 128))
    return loss[0, 0]
